# baseline (speedup 1.0000x reference)
import functools

import jax
import jax.numpy as jnp
from jax import lax
from jax.experimental import pallas as pl
from jax.experimental.pallas import tpu as pltpu
from jax.experimental.pallas import tpu_sc as plsc

N = 10000
D_FEAT = 128
HIDDEN = 16
N_CLASSES = 64
N_EDGES = 320000

N_PAD = 10240
PAD_ROW = N
CHUNK = 128
NCORES = 1
NSUB = 16
NW = NCORES * NSUB
E_TOT = N_EDGES + N
CPW = -(-E_TOT // (NW * CHUNK))
E_PAD = CPW * NW * CHUNK
ROWS_PER_SUB = N_PAD // NSUB

_mesh = plsc.VectorSubcoreMesh(core_axis_name="c", subcore_axis_name="s",
                               num_cores=NCORES)
_sc_params = pltpu.CompilerParams(use_tc_tiling_on_sc=False)


@jax.jit
def _sc_degree(dst2d, zeros_init, ones_blk):
    @functools.partial(
        pl.kernel,
        out_type=jax.ShapeDtypeStruct((NCORES, N_PAD, HIDDEN), jnp.float32),
        mesh=_mesh,
        scratch_types=[
            pltpu.VMEM((CPW, CHUNK), jnp.int32),
            pltpu.VMEM((CHUNK, HIDDEN), jnp.float32),
            pltpu.VMEM_SHARED((N_PAD, HIDDEN), jnp.float32),
            pltpu.SemaphoreType.DMA,
        ],
        compiler_params=_sc_params,
    )
    def k(dst_hbm, zero_hbm, ones_hbm, out_hbm, dst_v, ones_v, acc_sh, sem):
        cid = lax.axis_index("c")
        sid = lax.axis_index("s")
        wid = sid * NCORES + cid
        r0 = sid * ROWS_PER_SUB
        pltpu.sync_copy(zero_hbm.at[pl.ds(r0, ROWS_PER_SUB)],
                        acc_sh.at[pl.ds(r0, ROWS_PER_SUB)])
        pltpu.sync_copy(ones_hbm, ones_v)
        pltpu.sync_copy(dst_hbm.at[wid], dst_v)
        plsc.subcore_barrier()

        @pl.loop(0, CPW)
        def _(j):
            pltpu.sync_copy(ones_v, acc_sh.at[dst_v.at[j]], add=True)

        plsc.subcore_barrier()
        pltpu.sync_copy(acc_sh.at[pl.ds(r0, ROWS_PER_SUB)],
                        out_hbm.at[cid, pl.ds(r0, ROWS_PER_SUB)])

    return k(dst2d, zeros_init, ones_blk)


@jax.jit
def _sc_aggregate(hs, src2d, dst2d, zeros_init):
    @functools.partial(
        pl.kernel,
        out_type=jax.ShapeDtypeStruct((NCORES, N_PAD, HIDDEN), jnp.float32),
        mesh=_mesh,
        scratch_types=[
            pltpu.VMEM((CPW, CHUNK), jnp.int32),
            pltpu.VMEM((CPW, CHUNK), jnp.int32),
            pltpu.VMEM((CHUNK, HIDDEN), jnp.float32),
            pltpu.VMEM_SHARED((N_PAD, HIDDEN), jnp.float32),
            pltpu.VMEM_SHARED((N_PAD, HIDDEN), jnp.float32),
            pltpu.SemaphoreType.DMA,
        ],
        compiler_params=_sc_params,
    )
    def k(hs_hbm, src_hbm, dst_hbm, zero_hbm, out_hbm,
          src_v, dst_v, rows_v, acc_sh, tab_sh, sem):
        cid = lax.axis_index("c")
        sid = lax.axis_index("s")
        wid = sid * NCORES + cid
        r0 = sid * ROWS_PER_SUB
        pltpu.sync_copy(zero_hbm.at[pl.ds(r0, ROWS_PER_SUB)],
                        acc_sh.at[pl.ds(r0, ROWS_PER_SUB)])
        pltpu.sync_copy(hs_hbm.at[pl.ds(r0, ROWS_PER_SUB)],
                        tab_sh.at[pl.ds(r0, ROWS_PER_SUB)])
        pltpu.sync_copy(src_hbm.at[wid], src_v)
        pltpu.sync_copy(dst_hbm.at[wid], dst_v)
        plsc.subcore_barrier()

        @pl.loop(0, CPW)
        def _(j):
            pltpu.async_copy(tab_sh.at[src_v.at[j]], rows_v, sem).wait()
            pltpu.sync_copy(rows_v, acc_sh.at[dst_v.at[j]], add=True)

        plsc.subcore_barrier()
        pltpu.sync_copy(acc_sh.at[pl.ds(r0, ROWS_PER_SUB)],
                        out_hbm.at[cid, pl.ds(r0, ROWS_PER_SUB)])

    return k(hs, src2d, dst2d, zeros_init)


_BLK = 512
_GRID = N_PAD // _BLK


def _tc_prep(x_pad, W1, degP):
    def body(x_ref, w_ref, p_ref, hs_ref, dinv_ref):
        deg = sum(p_ref[c, :, :1] for c in range(NCORES))
        dinv = jnp.where(deg > 0, lax.rsqrt(deg), 0.0)
        h = jnp.dot(x_ref[...], w_ref[...], preferred_element_type=jnp.float32)
        hs_ref[...] = h * dinv
        dinv_ref[...] = jnp.broadcast_to(dinv, (_BLK, HIDDEN))

    return pl.pallas_call(
        body,
        grid=(_GRID,),
        in_specs=[
            pl.BlockSpec((_BLK, D_FEAT), lambda i: (i, 0)),
            pl.BlockSpec((D_FEAT, HIDDEN), lambda i: (0, 0)),
            pl.BlockSpec((NCORES, _BLK, HIDDEN), lambda i: (0, i, 0)),
        ],
        out_specs=[
            pl.BlockSpec((_BLK, HIDDEN), lambda i: (i, 0)),
            pl.BlockSpec((_BLK, HIDDEN), lambda i: (i, 0)),
        ],
        out_shape=[
            jax.ShapeDtypeStruct((N_PAD, HIDDEN), jnp.float32),
            jax.ShapeDtypeStruct((N_PAD, HIDDEN), jnp.float32),
        ],
    )(x_pad, W1, degP)


def _tc_mid(P1, dinv, b1):
    def body(p_ref, d_ref, b_ref, o_ref):
        agg = sum(p_ref[c] for c in range(NCORES))
        dinv_v = d_ref[...]
        rh = jnp.maximum(dinv_v * agg + b_ref[...], 0.0)
        o_ref[...] = dinv_v * rh

    return pl.pallas_call(
        body,
        grid=(_GRID,),
        in_specs=[
            pl.BlockSpec((NCORES, _BLK, HIDDEN), lambda i: (0, i, 0)),
            pl.BlockSpec((_BLK, HIDDEN), lambda i: (i, 0)),
            pl.BlockSpec((1, HIDDEN), lambda i: (0, 0)),
        ],
        out_specs=pl.BlockSpec((_BLK, HIDDEN), lambda i: (i, 0)),
        out_shape=jax.ShapeDtypeStruct((N_PAD, HIDDEN), jnp.float32),
    )(P1, dinv, b1)


def _tc_final(P2, dinv, W2, b2):
    def body(p_ref, d_ref, w_ref, b_ref, o_ref):
        agg = sum(p_ref[c] for c in range(NCORES)) * d_ref[...]
        z = jnp.dot(agg, w_ref[...], preferred_element_type=jnp.float32)
        z = z + b_ref[...]
        m = jnp.max(z, axis=1, keepdims=True)
        e = jnp.exp(z - m)
        lse = jnp.log(jnp.sum(e, axis=1, keepdims=True)) + m
        o_ref[...] = z - lse

    return pl.pallas_call(
        body,
        grid=(_GRID,),
        in_specs=[
            pl.BlockSpec((NCORES, _BLK, HIDDEN), lambda i: (0, i, 0)),
            pl.BlockSpec((_BLK, HIDDEN), lambda i: (i, 0)),
            pl.BlockSpec((HIDDEN, N_CLASSES), lambda i: (0, 0)),
            pl.BlockSpec((1, N_CLASSES), lambda i: (0, 0)),
        ],
        out_specs=pl.BlockSpec((_BLK, N_CLASSES), lambda i: (i, 0)),
        out_shape=jax.ShapeDtypeStruct((N_PAD, N_CLASSES), jnp.float32),
    )(P2, dinv, W2, b2)


def kernel(x, edge_index, W1, b1, W2, b2):
    src = edge_index[0].astype(jnp.int32)
    dst = edge_index[1].astype(jnp.int32)
    loop = jnp.arange(N, dtype=jnp.int32)
    pad = jnp.full((E_PAD - E_TOT,), PAD_ROW, dtype=jnp.int32)
    src2d = jnp.concatenate([src, loop, pad]).reshape(NW, CPW, CHUNK)
    dst2d = jnp.concatenate([dst, loop, pad]).reshape(NW, CPW, CHUNK)
    zeros_init = jnp.zeros((N_PAD, HIDDEN), jnp.float32)
    ones_blk = jnp.ones((CHUNK, HIDDEN), jnp.float32)
    x_pad = jnp.pad(x, ((0, N_PAD - N), (0, 0)))

    degP = _sc_degree(dst2d, zeros_init, ones_blk)
    hs1, dinv = _tc_prep(x_pad, W1, degP)
    P1 = _sc_aggregate(hs1, src2d, dst2d, zeros_init)
    hs2 = _tc_mid(P1, dinv, b1.reshape(1, HIDDEN))
    P2 = _sc_aggregate(hs2, src2d, dst2d, zeros_init)
    out = _tc_final(P2, dinv, W2, b2.reshape(1, N_CLASSES))
    return out[:N]


# speedup vs baseline: 1.4087x; 1.4087x over previous
import functools

import jax
import jax.numpy as jnp
from jax import lax
from jax.experimental import pallas as pl
from jax.experimental.pallas import tpu as pltpu
from jax.experimental.pallas import tpu_sc as plsc

N = 10000
D_FEAT = 128
HIDDEN = 16
N_CLASSES = 64
N_EDGES = 320000

CHUNK = 128
CHUNKS_TOT = N_EDGES // CHUNK
NCORES = 2
NSUB = 16
NW = NCORES * NSUB
MAX_CPW = -(-CHUNKS_TOT // NW) + 1
ROWS_PER_SUB = N // NSUB

_mesh = plsc.VectorSubcoreMesh(core_axis_name="c", subcore_axis_name="s",
                               num_cores=NCORES)
_sc_params = pltpu.CompilerParams(use_tc_tiling_on_sc=False)


@jax.jit
def _sc_degree(dst2d, zeros_init, ones_blk):
    @functools.partial(
        pl.kernel,
        out_type=jax.ShapeDtypeStruct((NCORES, N, HIDDEN), jnp.float32),
        mesh=_mesh,
        scratch_types=[
            pltpu.VMEM((MAX_CPW, CHUNK), jnp.int32),
            pltpu.VMEM((CHUNK, HIDDEN), jnp.float32),
            pltpu.VMEM_SHARED((N, HIDDEN), jnp.float32),
        ],
        compiler_params=_sc_params,
    )
    def k(dst_hbm, zero_hbm, ones_hbm, out_hbm, dst_v, ones_v, acc_sh):
        cid = lax.axis_index("c")
        sid = lax.axis_index("s")
        wid = sid * NCORES + cid
        c0 = (wid * CHUNKS_TOT) // NW
        c1 = ((wid + 1) * CHUNKS_TOT) // NW
        r0 = sid * ROWS_PER_SUB
        pltpu.sync_copy(zero_hbm.at[pl.ds(r0, ROWS_PER_SUB)],
                        acc_sh.at[pl.ds(r0, ROWS_PER_SUB)])
        pltpu.sync_copy(ones_hbm, ones_v)
        pltpu.sync_copy(dst_hbm.at[pl.ds(c0, MAX_CPW)], dst_v)
        plsc.subcore_barrier()

        @pl.loop(0, c1 - c0)
        def _(j):
            pltpu.sync_copy(ones_v, acc_sh.at[dst_v.at[j]], add=True)

        plsc.subcore_barrier()
        pltpu.sync_copy(acc_sh.at[pl.ds(r0, ROWS_PER_SUB)],
                        out_hbm.at[cid, pl.ds(r0, ROWS_PER_SUB)])

    return k(dst2d, zeros_init, ones_blk)


@jax.jit
def _sc_aggregate(hs, src2d, dst2d, zeros_init):
    @functools.partial(
        pl.kernel,
        out_type=jax.ShapeDtypeStruct((NCORES, N, HIDDEN), jnp.float32),
        mesh=_mesh,
        scratch_types=[
            pltpu.VMEM((MAX_CPW, CHUNK), jnp.int32),
            pltpu.VMEM((MAX_CPW, CHUNK), jnp.int32),
            pltpu.VMEM((CHUNK, HIDDEN), jnp.float32),
            pltpu.VMEM_SHARED((N, HIDDEN), jnp.float32),
            pltpu.VMEM_SHARED((N, HIDDEN), jnp.float32),
            pltpu.SemaphoreType.DMA,
        ],
        compiler_params=_sc_params,
    )
    def k(hs_hbm, src_hbm, dst_hbm, zero_hbm, out_hbm,
          src_v, dst_v, rows_v, acc_sh, tab_sh, sem):
        cid = lax.axis_index("c")
        sid = lax.axis_index("s")
        wid = sid * NCORES + cid
        c0 = (wid * CHUNKS_TOT) // NW
        c1 = ((wid + 1) * CHUNKS_TOT) // NW
        r0 = sid * ROWS_PER_SUB
        pltpu.sync_copy(zero_hbm.at[pl.ds(r0, ROWS_PER_SUB)],
                        acc_sh.at[pl.ds(r0, ROWS_PER_SUB)])
        pltpu.sync_copy(hs_hbm.at[pl.ds(r0, ROWS_PER_SUB)],
                        tab_sh.at[pl.ds(r0, ROWS_PER_SUB)])
        pltpu.sync_copy(src_hbm.at[pl.ds(c0, MAX_CPW)], src_v)
        pltpu.sync_copy(dst_hbm.at[pl.ds(c0, MAX_CPW)], dst_v)
        plsc.subcore_barrier()

        @pl.loop(0, c1 - c0)
        def _(j):
            pltpu.async_copy(tab_sh.at[src_v.at[j]], rows_v, sem).wait()
            pltpu.sync_copy(rows_v, acc_sh.at[dst_v.at[j]], add=True)

        plsc.subcore_barrier()
        pltpu.sync_copy(acc_sh.at[pl.ds(r0, ROWS_PER_SUB)],
                        out_hbm.at[cid, pl.ds(r0, ROWS_PER_SUB)])

    return k(hs, src2d, dst2d, zeros_init)


_BLK = 1000
_GRID = N // _BLK


def _tc_prep(x, W1, degP):
    def body(x_ref, w_ref, p_ref, hs_ref, dinv_ref):
        deg = sum(p_ref[c, :, :1] for c in range(NCORES)) + 1.0
        dinv = lax.rsqrt(deg)
        h = jnp.dot(x_ref[...], w_ref[...], preferred_element_type=jnp.float32)
        hs_ref[...] = h * dinv
        dinv_ref[...] = jnp.broadcast_to(dinv, (_BLK, HIDDEN))

    return pl.pallas_call(
        body,
        grid=(_GRID,),
        in_specs=[
            pl.BlockSpec((_BLK, D_FEAT), lambda i: (i, 0)),
            pl.BlockSpec((D_FEAT, HIDDEN), lambda i: (0, 0)),
            pl.BlockSpec((NCORES, _BLK, HIDDEN), lambda i: (0, i, 0)),
        ],
        out_specs=[
            pl.BlockSpec((_BLK, HIDDEN), lambda i: (i, 0)),
            pl.BlockSpec((_BLK, HIDDEN), lambda i: (i, 0)),
        ],
        out_shape=[
            jax.ShapeDtypeStruct((N, HIDDEN), jnp.float32),
            jax.ShapeDtypeStruct((N, HIDDEN), jnp.float32),
        ],
    )(x, W1, degP)


def _tc_mid(P1, hs1, dinv, b1):
    def body(p_ref, h_ref, d_ref, b_ref, o_ref):
        agg = sum(p_ref[c] for c in range(NCORES)) + h_ref[...]
        dinv_v = d_ref[...]
        rh = jnp.maximum(dinv_v * agg + b_ref[...], 0.0)
        o_ref[...] = dinv_v * rh

    return pl.pallas_call(
        body,
        grid=(_GRID,),
        in_specs=[
            pl.BlockSpec((NCORES, _BLK, HIDDEN), lambda i: (0, i, 0)),
            pl.BlockSpec((_BLK, HIDDEN), lambda i: (i, 0)),
            pl.BlockSpec((_BLK, HIDDEN), lambda i: (i, 0)),
            pl.BlockSpec((1, HIDDEN), lambda i: (0, 0)),
        ],
        out_specs=pl.BlockSpec((_BLK, HIDDEN), lambda i: (i, 0)),
        out_shape=jax.ShapeDtypeStruct((N, HIDDEN), jnp.float32),
    )(P1, hs1, dinv, b1)


def _tc_final(P2, hs2, dinv, W2, b2):
    def body(p_ref, h_ref, d_ref, w_ref, b_ref, o_ref):
        agg = sum(p_ref[c] for c in range(NCORES)) + h_ref[...]
        z = jnp.dot(agg * d_ref[...], w_ref[...],
                    preferred_element_type=jnp.float32)
        z = z + b_ref[...]
        m = jnp.max(z, axis=1, keepdims=True)
        e = jnp.exp(z - m)
        lse = jnp.log(jnp.sum(e, axis=1, keepdims=True)) + m
        o_ref[...] = z - lse

    return pl.pallas_call(
        body,
        grid=(_GRID,),
        in_specs=[
            pl.BlockSpec((NCORES, _BLK, HIDDEN), lambda i: (0, i, 0)),
            pl.BlockSpec((_BLK, HIDDEN), lambda i: (i, 0)),
            pl.BlockSpec((_BLK, HIDDEN), lambda i: (i, 0)),
            pl.BlockSpec((HIDDEN, N_CLASSES), lambda i: (0, 0)),
            pl.BlockSpec((1, N_CLASSES), lambda i: (0, 0)),
        ],
        out_specs=pl.BlockSpec((_BLK, N_CLASSES), lambda i: (i, 0)),
        out_shape=jax.ShapeDtypeStruct((N, N_CLASSES), jnp.float32),
    )(P2, hs2, dinv, W2, b2)


def kernel(x, edge_index, W1, b1, W2, b2):
    src2d = edge_index[0].astype(jnp.int32).reshape(CHUNKS_TOT, CHUNK)
    dst2d = edge_index[1].astype(jnp.int32).reshape(CHUNKS_TOT, CHUNK)
    zeros_init = jnp.zeros((N, HIDDEN), jnp.float32)
    ones_blk = jnp.ones((CHUNK, HIDDEN), jnp.float32)

    degP = _sc_degree(dst2d, zeros_init, ones_blk)
    hs1, dinv = _tc_prep(x, W1, degP)
    P1 = _sc_aggregate(hs1, src2d, dst2d, zeros_init)
    hs2 = _tc_mid(P1, hs1, dinv, b1.reshape(1, HIDDEN))
    P2 = _sc_aggregate(hs2, src2d, dst2d, zeros_init)
    return _tc_final(P2, hs2, dinv, W2, b2.reshape(1, N_CLASSES))


# speedup vs baseline: 1.5906x; 1.1291x over previous
import functools

import jax
import jax.numpy as jnp
from jax import lax
from jax.experimental import pallas as pl
from jax.experimental.pallas import tpu as pltpu
from jax.experimental.pallas import tpu_sc as plsc

N = 10000
D_FEAT = 128
HIDDEN = 16
N_CLASSES = 64
N_EDGES = 320000

CHUNK = 128
CHUNKS_TOT = N_EDGES // CHUNK
NCORES = 2
NSUB = 16
NW = NCORES * NSUB
MAX_CPW = -(-CHUNKS_TOT // NW) + 1
ROWS_PER_SUB = N // NSUB

_mesh = plsc.VectorSubcoreMesh(core_axis_name="c", subcore_axis_name="s",
                               num_cores=NCORES)
_sc_params = pltpu.CompilerParams(use_tc_tiling_on_sc=False)


@jax.jit
def _sc_degree(dst2d, zeros_init, ones_blk):
    @functools.partial(
        pl.kernel,
        out_type=jax.ShapeDtypeStruct((NCORES, N, HIDDEN), jnp.float32),
        mesh=_mesh,
        scratch_types=[
            pltpu.VMEM((MAX_CPW, CHUNK), jnp.int32),
            pltpu.VMEM((CHUNK, HIDDEN), jnp.float32),
            pltpu.VMEM_SHARED((N, HIDDEN), jnp.float32),
            pltpu.SemaphoreType.DMA,
        ],
        compiler_params=_sc_params,
    )
    def k(dst_hbm, zero_hbm, ones_hbm, out_hbm, dst_v, ones_v, acc_sh, sem):
        cid = lax.axis_index("c")
        sid = lax.axis_index("s")
        wid = sid * NCORES + cid
        c0 = (wid * CHUNKS_TOT) // NW
        c1 = ((wid + 1) * CHUNKS_TOT) // NW
        r0 = sid * ROWS_PER_SUB
        pltpu.sync_copy(zero_hbm.at[pl.ds(r0, ROWS_PER_SUB)],
                        acc_sh.at[pl.ds(r0, ROWS_PER_SUB)])
        pltpu.sync_copy(ones_hbm, ones_v)
        pltpu.sync_copy(dst_hbm.at[pl.ds(c0, MAX_CPW)], dst_v)
        plsc.subcore_barrier()

        nch = c1 - c0

        @pl.loop(0, nch)
        def _(j):
            pltpu.async_copy(ones_v, acc_sh.at[dst_v.at[j]], sem, add=True)

        @pl.loop(0, nch)
        def _(j):
            pltpu.make_async_copy(ones_v, acc_sh.at[dst_v.at[j]], sem).wait()

        plsc.subcore_barrier()
        pltpu.sync_copy(acc_sh.at[pl.ds(r0, ROWS_PER_SUB)],
                        out_hbm.at[cid, pl.ds(r0, ROWS_PER_SUB)])

    return k(dst2d, zeros_init, ones_blk)


@jax.jit
def _sc_aggregate(hs, src2d, dst2d, zeros_init):
    @functools.partial(
        pl.kernel,
        out_type=jax.ShapeDtypeStruct((NCORES, N, HIDDEN), jnp.float32),
        mesh=_mesh,
        scratch_types=[
            pltpu.VMEM((MAX_CPW, CHUNK), jnp.int32),
            pltpu.VMEM((MAX_CPW, CHUNK), jnp.int32),
            pltpu.VMEM((2, CHUNK, HIDDEN), jnp.float32),
            pltpu.VMEM_SHARED((N, HIDDEN), jnp.float32),
            pltpu.VMEM_SHARED((N, HIDDEN), jnp.float32),
            pltpu.SemaphoreType.DMA,
        ],
        compiler_params=_sc_params,
    )
    def k(hs_hbm, src_hbm, dst_hbm, zero_hbm, out_hbm,
          src_v, dst_v, rows_v, acc_sh, tab_sh, sem):
        cid = lax.axis_index("c")
        sid = lax.axis_index("s")
        wid = sid * NCORES + cid
        c0 = (wid * CHUNKS_TOT) // NW
        c1 = ((wid + 1) * CHUNKS_TOT) // NW
        r0 = sid * ROWS_PER_SUB
        pltpu.sync_copy(zero_hbm.at[pl.ds(r0, ROWS_PER_SUB)],
                        acc_sh.at[pl.ds(r0, ROWS_PER_SUB)])
        pltpu.sync_copy(hs_hbm.at[pl.ds(r0, ROWS_PER_SUB)],
                        tab_sh.at[pl.ds(r0, ROWS_PER_SUB)])
        pltpu.sync_copy(src_hbm.at[pl.ds(c0, MAX_CPW)], src_v)
        pltpu.sync_copy(dst_hbm.at[pl.ds(c0, MAX_CPW)], dst_v)
        plsc.subcore_barrier()

        nch = c1 - c0
        pltpu.async_copy(tab_sh.at[src_v.at[0]], rows_v.at[0], sem)

        @pl.loop(0, nch)
        def _(j):
            @pl.when(j + 1 < nch)
            def _():
                @pl.when(lax.rem(j, 2) == 0)
                def _():
                    pltpu.async_copy(tab_sh.at[src_v.at[j + 1]],
                                     rows_v.at[1], sem)

                @pl.when(lax.rem(j, 2) == 1)
                def _():
                    pltpu.async_copy(tab_sh.at[src_v.at[j + 1]],
                                     rows_v.at[0], sem)

            pltpu.make_async_copy(tab_sh.at[src_v.at[j]],
                                  rows_v.at[0], sem).wait()

            @pl.when(lax.rem(j, 2) == 0)
            def _():
                pltpu.sync_copy(rows_v.at[0], acc_sh.at[dst_v.at[j]], add=True)

            @pl.when(lax.rem(j, 2) == 1)
            def _():
                pltpu.sync_copy(rows_v.at[1], acc_sh.at[dst_v.at[j]], add=True)

        plsc.subcore_barrier()
        pltpu.sync_copy(acc_sh.at[pl.ds(r0, ROWS_PER_SUB)],
                        out_hbm.at[cid, pl.ds(r0, ROWS_PER_SUB)])

    return k(hs, src2d, dst2d, zeros_init)


_BLK = 1000
_GRID = N // _BLK


def _tc_prep(x, W1, degP):
    def body(x_ref, w_ref, p_ref, hs_ref, dinv_ref):
        deg = sum(p_ref[c, :, :1] for c in range(NCORES)) + 1.0
        dinv = lax.rsqrt(deg)
        h = jnp.dot(x_ref[...], w_ref[...], preferred_element_type=jnp.float32)
        hs_ref[...] = h * dinv
        dinv_ref[...] = jnp.broadcast_to(dinv, (_BLK, HIDDEN))

    return pl.pallas_call(
        body,
        grid=(_GRID,),
        in_specs=[
            pl.BlockSpec((_BLK, D_FEAT), lambda i: (i, 0)),
            pl.BlockSpec((D_FEAT, HIDDEN), lambda i: (0, 0)),
            pl.BlockSpec((NCORES, _BLK, HIDDEN), lambda i: (0, i, 0)),
        ],
        out_specs=[
            pl.BlockSpec((_BLK, HIDDEN), lambda i: (i, 0)),
            pl.BlockSpec((_BLK, HIDDEN), lambda i: (i, 0)),
        ],
        out_shape=[
            jax.ShapeDtypeStruct((N, HIDDEN), jnp.float32),
            jax.ShapeDtypeStruct((N, HIDDEN), jnp.float32),
        ],
    )(x, W1, degP)


def _tc_mid(P1, hs1, dinv, b1):
    def body(p_ref, h_ref, d_ref, b_ref, o_ref):
        agg = sum(p_ref[c] for c in range(NCORES)) + h_ref[...]
        dinv_v = d_ref[...]
        rh = jnp.maximum(dinv_v * agg + b_ref[...], 0.0)
        o_ref[...] = dinv_v * rh

    return pl.pallas_call(
        body,
        grid=(_GRID,),
        in_specs=[
            pl.BlockSpec((NCORES, _BLK, HIDDEN), lambda i: (0, i, 0)),
            pl.BlockSpec((_BLK, HIDDEN), lambda i: (i, 0)),
            pl.BlockSpec((_BLK, HIDDEN), lambda i: (i, 0)),
            pl.BlockSpec((1, HIDDEN), lambda i: (0, 0)),
        ],
        out_specs=pl.BlockSpec((_BLK, HIDDEN), lambda i: (i, 0)),
        out_shape=jax.ShapeDtypeStruct((N, HIDDEN), jnp.float32),
    )(P1, hs1, dinv, b1)


def _tc_final(P2, hs2, dinv, W2, b2):
    def body(p_ref, h_ref, d_ref, w_ref, b_ref, o_ref):
        agg = sum(p_ref[c] for c in range(NCORES)) + h_ref[...]
        z = jnp.dot(agg * d_ref[...], w_ref[...],
                    preferred_element_type=jnp.float32)
        z = z + b_ref[...]
        m = jnp.max(z, axis=1, keepdims=True)
        e = jnp.exp(z - m)
        lse = jnp.log(jnp.sum(e, axis=1, keepdims=True)) + m
        o_ref[...] = z - lse

    return pl.pallas_call(
        body,
        grid=(_GRID,),
        in_specs=[
            pl.BlockSpec((NCORES, _BLK, HIDDEN), lambda i: (0, i, 0)),
            pl.BlockSpec((_BLK, HIDDEN), lambda i: (i, 0)),
            pl.BlockSpec((_BLK, HIDDEN), lambda i: (i, 0)),
            pl.BlockSpec((HIDDEN, N_CLASSES), lambda i: (0, 0)),
            pl.BlockSpec((1, N_CLASSES), lambda i: (0, 0)),
        ],
        out_specs=pl.BlockSpec((_BLK, N_CLASSES), lambda i: (i, 0)),
        out_shape=jax.ShapeDtypeStruct((N, N_CLASSES), jnp.float32),
    )(P2, hs2, dinv, W2, b2)


def kernel(x, edge_index, W1, b1, W2, b2):
    src2d = edge_index[0].astype(jnp.int32).reshape(CHUNKS_TOT, CHUNK)
    dst2d = edge_index[1].astype(jnp.int32).reshape(CHUNKS_TOT, CHUNK)
    zeros_init = jnp.zeros((N, HIDDEN), jnp.float32)
    ones_blk = jnp.ones((CHUNK, HIDDEN), jnp.float32)

    degP = _sc_degree(dst2d, zeros_init, ones_blk)
    hs1, dinv = _tc_prep(x, W1, degP)
    P1 = _sc_aggregate(hs1, src2d, dst2d, zeros_init)
    hs2 = _tc_mid(P1, hs1, dinv, b1.reshape(1, HIDDEN))
    P2 = _sc_aggregate(hs2, src2d, dst2d, zeros_init)
    return _tc_final(P2, hs2, dinv, W2, b2.reshape(1, N_CLASSES))


# speedup vs baseline: 1.7662x; 1.1104x over previous
import functools

import jax
import jax.numpy as jnp
from jax import lax
from jax.experimental import pallas as pl
from jax.experimental.pallas import tpu as pltpu
from jax.experimental.pallas import tpu_sc as plsc

N = 10000
D_FEAT = 128
HIDDEN = 16
N_CLASSES = 64
N_EDGES = 320000

CHUNK = 128
CHUNKS_TOT = N_EDGES // CHUNK
NCORES = 2
NSUB = 16
NW = NCORES * NSUB
MAX_CPW = -(-CHUNKS_TOT // NW) + 1
ROWS_PER_SUB = N // NSUB

_mesh = plsc.VectorSubcoreMesh(core_axis_name="c", subcore_axis_name="s",
                               num_cores=NCORES)
_sc_params = pltpu.CompilerParams(use_tc_tiling_on_sc=False)


@jax.jit
def _sc_degree(e32, zeros_init, ones_blk):
    @functools.partial(
        pl.kernel,
        out_type=jax.ShapeDtypeStruct((NCORES, N, HIDDEN), jnp.float32),
        mesh=_mesh,
        scratch_types=[
            pltpu.VMEM((MAX_CPW * CHUNK,), jnp.int32),
            pltpu.VMEM((CHUNK, HIDDEN), jnp.float32),
            pltpu.VMEM_SHARED((N, HIDDEN), jnp.float32),
            pltpu.SemaphoreType.DMA,
        ],
        compiler_params=_sc_params,
    )
    def k(e_hbm, zero_hbm, ones_hbm, out_hbm, dst_v, ones_v, acc_sh, sem):
        cid = lax.axis_index("c")
        sid = lax.axis_index("s")
        wid = sid * NCORES + cid
        c0 = (wid * CHUNKS_TOT) // NW
        c1 = ((wid + 1) * CHUNKS_TOT) // NW
        r0 = sid * ROWS_PER_SUB
        pltpu.sync_copy(zero_hbm.at[pl.ds(r0, ROWS_PER_SUB)],
                        acc_sh.at[pl.ds(r0, ROWS_PER_SUB)])
        pltpu.sync_copy(ones_hbm, ones_v)
        pltpu.sync_copy(e_hbm.at[1, pl.ds(c0 * CHUNK, MAX_CPW * CHUNK)], dst_v)
        plsc.subcore_barrier()

        nch = c1 - c0

        @pl.loop(0, nch)
        def _(j):
            pltpu.async_copy(ones_v,
                             acc_sh.at[dst_v.at[pl.ds(j * CHUNK, CHUNK)]],
                             sem, add=True)

        @pl.loop(0, nch)
        def _(j):
            pltpu.make_async_copy(
                ones_v, acc_sh.at[dst_v.at[pl.ds(j * CHUNK, CHUNK)]],
                sem).wait()

        plsc.subcore_barrier()
        pltpu.sync_copy(acc_sh.at[pl.ds(r0, ROWS_PER_SUB)],
                        out_hbm.at[cid, pl.ds(r0, ROWS_PER_SUB)])

    return k(e32, zeros_init, ones_blk)


@jax.jit
def _sc_aggregate(hs, e32, zeros_init):
    @functools.partial(
        pl.kernel,
        out_type=jax.ShapeDtypeStruct((NCORES, N, HIDDEN), jnp.float32),
        mesh=_mesh,
        scratch_types=[
            pltpu.VMEM((MAX_CPW * CHUNK,), jnp.int32),
            pltpu.VMEM((MAX_CPW * CHUNK,), jnp.int32),
            pltpu.VMEM((2, CHUNK, HIDDEN), jnp.float32),
            pltpu.VMEM_SHARED((N, HIDDEN), jnp.float32),
            pltpu.VMEM_SHARED((N, HIDDEN), jnp.float32),
            pltpu.SemaphoreType.DMA,
        ],
        compiler_params=_sc_params,
    )
    def k(hs_hbm, e_hbm, zero_hbm, out_hbm,
          src_v, dst_v, rows_v, acc_sh, tab_sh, sem):
        cid = lax.axis_index("c")
        sid = lax.axis_index("s")
        wid = sid * NCORES + cid
        c0 = (wid * CHUNKS_TOT) // NW
        c1 = ((wid + 1) * CHUNKS_TOT) // NW
        r0 = sid * ROWS_PER_SUB
        pltpu.sync_copy(zero_hbm.at[pl.ds(r0, ROWS_PER_SUB)],
                        acc_sh.at[pl.ds(r0, ROWS_PER_SUB)])
        pltpu.sync_copy(hs_hbm.at[pl.ds(r0, ROWS_PER_SUB)],
                        tab_sh.at[pl.ds(r0, ROWS_PER_SUB)])
        pltpu.sync_copy(e_hbm.at[0, pl.ds(c0 * CHUNK, MAX_CPW * CHUNK)], src_v)
        pltpu.sync_copy(e_hbm.at[1, pl.ds(c0 * CHUNK, MAX_CPW * CHUNK)], dst_v)
        plsc.subcore_barrier()

        nch = c1 - c0

        def src_at(j):
            return src_v.at[pl.ds(j * CHUNK, CHUNK)]

        def dst_at(j):
            return dst_v.at[pl.ds(j * CHUNK, CHUNK)]

        pltpu.async_copy(tab_sh.at[src_at(0)], rows_v.at[0], sem)

        @pl.loop(0, nch)
        def _(j):
            @pl.when(j + 1 < nch)
            def _():
                @pl.when(lax.rem(j, 2) == 0)
                def _():
                    pltpu.async_copy(tab_sh.at[src_at(j + 1)],
                                     rows_v.at[1], sem)

                @pl.when(lax.rem(j, 2) == 1)
                def _():
                    pltpu.async_copy(tab_sh.at[src_at(j + 1)],
                                     rows_v.at[0], sem)

            pltpu.make_async_copy(tab_sh.at[src_at(j)],
                                  rows_v.at[0], sem).wait()

            @pl.when(lax.rem(j, 2) == 0)
            def _():
                pltpu.sync_copy(rows_v.at[0], acc_sh.at[dst_at(j)], add=True)

            @pl.when(lax.rem(j, 2) == 1)
            def _():
                pltpu.sync_copy(rows_v.at[1], acc_sh.at[dst_at(j)], add=True)

        plsc.subcore_barrier()
        pltpu.sync_copy(acc_sh.at[pl.ds(r0, ROWS_PER_SUB)],
                        out_hbm.at[cid, pl.ds(r0, ROWS_PER_SUB)])

    return k(hs, e32, zeros_init)


_BLK = 10000
_GRID = N // _BLK


def _tc_prep(x, W1, degP):
    def body(x_ref, w_ref, p_ref, hs_ref, dinv_ref):
        deg = sum(p_ref[c, :, :1] for c in range(NCORES)) + 1.0
        dinv = lax.rsqrt(deg)
        h = jnp.dot(x_ref[...], w_ref[...], preferred_element_type=jnp.float32)
        hs_ref[...] = h * dinv
        dinv_ref[...] = jnp.broadcast_to(dinv, (_BLK, HIDDEN))

    return pl.pallas_call(
        body,
        grid=(_GRID,),
        in_specs=[
            pl.BlockSpec((_BLK, D_FEAT), lambda i: (i, 0)),
            pl.BlockSpec((D_FEAT, HIDDEN), lambda i: (0, 0)),
            pl.BlockSpec((NCORES, _BLK, HIDDEN), lambda i: (0, i, 0)),
        ],
        out_specs=[
            pl.BlockSpec((_BLK, HIDDEN), lambda i: (i, 0)),
            pl.BlockSpec((_BLK, HIDDEN), lambda i: (i, 0)),
        ],
        out_shape=[
            jax.ShapeDtypeStruct((N, HIDDEN), jnp.float32),
            jax.ShapeDtypeStruct((N, HIDDEN), jnp.float32),
        ],
    )(x, W1, degP)


def _tc_mid(P1, hs1, dinv, b1):
    def body(p_ref, h_ref, d_ref, b_ref, o_ref):
        agg = sum(p_ref[c] for c in range(NCORES)) + h_ref[...]
        dinv_v = d_ref[...]
        rh = jnp.maximum(dinv_v * agg + b_ref[...], 0.0)
        o_ref[...] = dinv_v * rh

    return pl.pallas_call(
        body,
        grid=(_GRID,),
        in_specs=[
            pl.BlockSpec((NCORES, _BLK, HIDDEN), lambda i: (0, i, 0)),
            pl.BlockSpec((_BLK, HIDDEN), lambda i: (i, 0)),
            pl.BlockSpec((_BLK, HIDDEN), lambda i: (i, 0)),
            pl.BlockSpec((1, HIDDEN), lambda i: (0, 0)),
        ],
        out_specs=pl.BlockSpec((_BLK, HIDDEN), lambda i: (i, 0)),
        out_shape=jax.ShapeDtypeStruct((N, HIDDEN), jnp.float32),
    )(P1, hs1, dinv, b1)


def _tc_final(P2, hs2, dinv, W2, b2):
    def body(p_ref, h_ref, d_ref, w_ref, b_ref, o_ref):
        agg = sum(p_ref[c] for c in range(NCORES)) + h_ref[...]
        z = jnp.dot(agg * d_ref[...], w_ref[...],
                    preferred_element_type=jnp.float32)
        z = z + b_ref[...]
        m = jnp.max(z, axis=1, keepdims=True)
        e = jnp.exp(z - m)
        lse = jnp.log(jnp.sum(e, axis=1, keepdims=True)) + m
        o_ref[...] = z - lse

    return pl.pallas_call(
        body,
        grid=(_GRID,),
        in_specs=[
            pl.BlockSpec((NCORES, _BLK, HIDDEN), lambda i: (0, i, 0)),
            pl.BlockSpec((_BLK, HIDDEN), lambda i: (i, 0)),
            pl.BlockSpec((_BLK, HIDDEN), lambda i: (i, 0)),
            pl.BlockSpec((HIDDEN, N_CLASSES), lambda i: (0, 0)),
            pl.BlockSpec((1, N_CLASSES), lambda i: (0, 0)),
        ],
        out_specs=pl.BlockSpec((_BLK, N_CLASSES), lambda i: (i, 0)),
        out_shape=jax.ShapeDtypeStruct((N, N_CLASSES), jnp.float32),
    )(P2, hs2, dinv, W2, b2)


def kernel(x, edge_index, W1, b1, W2, b2):
    e32 = edge_index.astype(jnp.int32)
    zeros_init = jnp.zeros((N, HIDDEN), jnp.float32)
    ones_blk = jnp.ones((CHUNK, HIDDEN), jnp.float32)

    degP = _sc_degree(e32, zeros_init, ones_blk)
    hs1, dinv = _tc_prep(x, W1, degP)
    P1 = _sc_aggregate(hs1, e32, zeros_init)
    hs2 = _tc_mid(P1, hs1, dinv, b1.reshape(1, HIDDEN))
    P2 = _sc_aggregate(hs2, e32, zeros_init)
    return _tc_final(P2, hs2, dinv, W2, b2.reshape(1, N_CLASSES))


# speedup vs baseline: 1.8099x; 1.0248x over previous
import functools

import jax
import jax.numpy as jnp
from jax import lax
from jax.experimental import pallas as pl
from jax.experimental.pallas import tpu as pltpu
from jax.experimental.pallas import tpu_sc as plsc

N = 10000
D_FEAT = 128
HIDDEN = 16
N_CLASSES = 64
N_EDGES = 320000

CHUNK = 128
CHUNKS_TOT = N_EDGES // CHUNK
NCORES = 2
NSUB = 16
NW = NCORES * NSUB
MAX_CPW = -(-CHUNKS_TOT // NW) + 1
ROWS_PER_SUB = N // NSUB

_mesh = plsc.VectorSubcoreMesh(core_axis_name="c", subcore_axis_name="s",
                               num_cores=NCORES)
_sc_params = pltpu.CompilerParams(use_tc_tiling_on_sc=False)


@jax.jit
def _sc_degree(e32, zeros_init, ones_blk):
    @functools.partial(
        pl.kernel,
        out_type=jax.ShapeDtypeStruct((NCORES, N, HIDDEN), jnp.float32),
        mesh=_mesh,
        scratch_types=[
            pltpu.VMEM((MAX_CPW * CHUNK,), jnp.int32),
            pltpu.VMEM((CHUNK, HIDDEN), jnp.float32),
            pltpu.VMEM_SHARED((N, HIDDEN), jnp.float32),
            pltpu.SemaphoreType.DMA,
        ],
        compiler_params=_sc_params,
    )
    def k(e_hbm, zero_hbm, ones_hbm, out_hbm, dst_v, ones_v, acc_sh, sem):
        cid = lax.axis_index("c")
        sid = lax.axis_index("s")
        wid = sid * NCORES + cid
        c0 = (wid * CHUNKS_TOT) // NW
        c1 = ((wid + 1) * CHUNKS_TOT) // NW
        r0 = sid * ROWS_PER_SUB
        pltpu.sync_copy(zero_hbm.at[pl.ds(r0, ROWS_PER_SUB)],
                        acc_sh.at[pl.ds(r0, ROWS_PER_SUB)])
        pltpu.sync_copy(ones_hbm, ones_v)
        pltpu.sync_copy(e_hbm.at[1, pl.ds(c0 * CHUNK, MAX_CPW * CHUNK)], dst_v)
        plsc.subcore_barrier()

        nch = c1 - c0

        @pl.loop(0, nch)
        def _(j):
            pltpu.async_copy(ones_v,
                             acc_sh.at[dst_v.at[pl.ds(j * CHUNK, CHUNK)]],
                             sem, add=True)

        @pl.loop(0, nch)
        def _(j):
            pltpu.make_async_copy(
                ones_v, acc_sh.at[dst_v.at[pl.ds(j * CHUNK, CHUNK)]],
                sem).wait()

        plsc.subcore_barrier()
        pltpu.sync_copy(acc_sh.at[pl.ds(r0, ROWS_PER_SUB)],
                        out_hbm.at[cid, pl.ds(r0, ROWS_PER_SUB)])

    return k(e32, zeros_init, ones_blk)


@jax.jit
def _sc_aggregate(hs, e32, zeros_init):
    @functools.partial(
        pl.kernel,
        out_type=jax.ShapeDtypeStruct((NCORES, N, HIDDEN), jnp.float32),
        mesh=_mesh,
        scratch_types=[
            pltpu.VMEM((MAX_CPW * CHUNK,), jnp.int32),
            pltpu.VMEM((MAX_CPW * CHUNK,), jnp.int32),
            pltpu.VMEM((2, CHUNK, HIDDEN), jnp.float32),
            pltpu.VMEM_SHARED((N, HIDDEN), jnp.float32),
            pltpu.VMEM_SHARED((N, HIDDEN), jnp.float32),
            pltpu.SemaphoreType.DMA,
            pltpu.SemaphoreType.DMA,
        ],
        compiler_params=_sc_params,
    )
    def k(hs_hbm, e_hbm, zero_hbm, out_hbm,
          src_v, dst_v, rows_v, acc_sh, tab_sh, sem, sem_s):
        cid = lax.axis_index("c")
        sid = lax.axis_index("s")
        wid = sid * NCORES + cid
        c0 = (wid * CHUNKS_TOT) // NW
        c1 = ((wid + 1) * CHUNKS_TOT) // NW
        r0 = sid * ROWS_PER_SUB
        st = [
            pltpu.async_copy(zero_hbm.at[pl.ds(r0, ROWS_PER_SUB)],
                             acc_sh.at[pl.ds(r0, ROWS_PER_SUB)], sem),
            pltpu.async_copy(hs_hbm.at[pl.ds(r0, ROWS_PER_SUB)],
                             tab_sh.at[pl.ds(r0, ROWS_PER_SUB)], sem),
            pltpu.async_copy(e_hbm.at[0, pl.ds(c0 * CHUNK, MAX_CPW * CHUNK)],
                             src_v, sem),
            pltpu.async_copy(e_hbm.at[1, pl.ds(c0 * CHUNK, MAX_CPW * CHUNK)],
                             dst_v, sem),
        ]
        for d in st:
            d.wait()
        plsc.subcore_barrier()

        nch = c1 - c0

        def src_at(j):
            return src_v.at[pl.ds(j * CHUNK, CHUNK)]

        def dst_at(j):
            return dst_v.at[pl.ds(j * CHUNK, CHUNK)]

        pltpu.async_copy(tab_sh.at[src_at(0)], rows_v.at[0], sem)

        @pl.loop(0, nch)
        def _(j):
            @pl.when(j > 0)
            def _():
                @pl.when(lax.rem(j, 2) == 1)
                def _():
                    pltpu.make_async_copy(rows_v.at[0],
                                          acc_sh.at[dst_at(j - 1)],
                                          sem_s).wait()

                @pl.when(lax.rem(j, 2) == 0)
                def _():
                    pltpu.make_async_copy(rows_v.at[1],
                                          acc_sh.at[dst_at(j - 1)],
                                          sem_s).wait()

            @pl.when(j + 1 < nch)
            def _():
                @pl.when(lax.rem(j, 2) == 0)
                def _():
                    pltpu.async_copy(tab_sh.at[src_at(j + 1)],
                                     rows_v.at[1], sem)

                @pl.when(lax.rem(j, 2) == 1)
                def _():
                    pltpu.async_copy(tab_sh.at[src_at(j + 1)],
                                     rows_v.at[0], sem)

            pltpu.make_async_copy(tab_sh.at[src_at(j)],
                                  rows_v.at[0], sem).wait()

            @pl.when(lax.rem(j, 2) == 0)
            def _():
                pltpu.async_copy(rows_v.at[0], acc_sh.at[dst_at(j)],
                                 sem_s, add=True)

            @pl.when(lax.rem(j, 2) == 1)
            def _():
                pltpu.async_copy(rows_v.at[1], acc_sh.at[dst_at(j)],
                                 sem_s, add=True)

        pltpu.make_async_copy(rows_v.at[0], acc_sh.at[dst_at(nch - 1)],
                              sem_s).wait()
        plsc.subcore_barrier()
        pltpu.sync_copy(acc_sh.at[pl.ds(r0, ROWS_PER_SUB)],
                        out_hbm.at[cid, pl.ds(r0, ROWS_PER_SUB)])

    return k(hs, e32, zeros_init)


_BLK = 10000
_GRID = N // _BLK


def _tc_prep(x, W1, degP):
    def body(x_ref, w_ref, p_ref, hs_ref, dinv_ref):
        deg = sum(p_ref[c, :, :1] for c in range(NCORES)) + 1.0
        dinv = lax.rsqrt(deg)
        h = jnp.dot(x_ref[...], w_ref[...], preferred_element_type=jnp.float32)
        hs_ref[...] = h * dinv
        dinv_ref[...] = jnp.broadcast_to(dinv, (_BLK, HIDDEN))

    return pl.pallas_call(
        body,
        grid=(_GRID,),
        in_specs=[
            pl.BlockSpec((_BLK, D_FEAT), lambda i: (i, 0)),
            pl.BlockSpec((D_FEAT, HIDDEN), lambda i: (0, 0)),
            pl.BlockSpec((NCORES, _BLK, HIDDEN), lambda i: (0, i, 0)),
        ],
        out_specs=[
            pl.BlockSpec((_BLK, HIDDEN), lambda i: (i, 0)),
            pl.BlockSpec((_BLK, HIDDEN), lambda i: (i, 0)),
        ],
        out_shape=[
            jax.ShapeDtypeStruct((N, HIDDEN), jnp.float32),
            jax.ShapeDtypeStruct((N, HIDDEN), jnp.float32),
        ],
    )(x, W1, degP)


def _tc_mid(P1, hs1, dinv, b1):
    def body(p_ref, h_ref, d_ref, b_ref, o_ref):
        agg = sum(p_ref[c] for c in range(NCORES)) + h_ref[...]
        dinv_v = d_ref[...]
        rh = jnp.maximum(dinv_v * agg + b_ref[...], 0.0)
        o_ref[...] = dinv_v * rh

    return pl.pallas_call(
        body,
        grid=(_GRID,),
        in_specs=[
            pl.BlockSpec((NCORES, _BLK, HIDDEN), lambda i: (0, i, 0)),
            pl.BlockSpec((_BLK, HIDDEN), lambda i: (i, 0)),
            pl.BlockSpec((_BLK, HIDDEN), lambda i: (i, 0)),
            pl.BlockSpec((1, HIDDEN), lambda i: (0, 0)),
        ],
        out_specs=pl.BlockSpec((_BLK, HIDDEN), lambda i: (i, 0)),
        out_shape=jax.ShapeDtypeStruct((N, HIDDEN), jnp.float32),
    )(P1, hs1, dinv, b1)


def _tc_final(P2, hs2, dinv, W2, b2):
    def body(p_ref, h_ref, d_ref, w_ref, b_ref, o_ref):
        agg = sum(p_ref[c] for c in range(NCORES)) + h_ref[...]
        z = jnp.dot(agg * d_ref[...], w_ref[...],
                    preferred_element_type=jnp.float32)
        z = z + b_ref[...]
        m = jnp.max(z, axis=1, keepdims=True)
        e = jnp.exp(z - m)
        lse = jnp.log(jnp.sum(e, axis=1, keepdims=True)) + m
        o_ref[...] = z - lse

    return pl.pallas_call(
        body,
        grid=(_GRID,),
        in_specs=[
            pl.BlockSpec((NCORES, _BLK, HIDDEN), lambda i: (0, i, 0)),
            pl.BlockSpec((_BLK, HIDDEN), lambda i: (i, 0)),
            pl.BlockSpec((_BLK, HIDDEN), lambda i: (i, 0)),
            pl.BlockSpec((HIDDEN, N_CLASSES), lambda i: (0, 0)),
            pl.BlockSpec((1, N_CLASSES), lambda i: (0, 0)),
        ],
        out_specs=pl.BlockSpec((_BLK, N_CLASSES), lambda i: (i, 0)),
        out_shape=jax.ShapeDtypeStruct((N, N_CLASSES), jnp.float32),
    )(P2, hs2, dinv, W2, b2)


def kernel(x, edge_index, W1, b1, W2, b2):
    e32 = edge_index.astype(jnp.int32)
    zeros_init = jnp.zeros((N, HIDDEN), jnp.float32)
    ones_blk = jnp.ones((CHUNK, HIDDEN), jnp.float32)

    degP = _sc_degree(e32, zeros_init, ones_blk)
    hs1, dinv = _tc_prep(x, W1, degP)
    P1 = _sc_aggregate(hs1, e32, zeros_init)
    hs2 = _tc_mid(P1, hs1, dinv, b1.reshape(1, HIDDEN))
    P2 = _sc_aggregate(hs2, e32, zeros_init)
    return _tc_final(P2, hs2, dinv, W2, b2.reshape(1, N_CLASSES))
